# baseline (speedup 1.0000x reference)
import functools

import jax
import jax.numpy as jnp
from jax import lax
from jax.experimental import pallas as pl
from jax.experimental.pallas import tpu as pltpu
from jax.experimental.pallas import tpu_sc as plsc

_DIM = 256
_NE = 8192
_TOKENS = 16384
_MT = 128
_NBLK = _TOKENS // _MT


def _vq_body(f_ref, e_ref, fb_ref, eb_ref,
             idx_ref, loss_ref, perp_ref, counts_ref, acc_ref):
    i = pl.program_id(0)

    @pl.when(i == 0)
    def _init():
        counts_ref[...] = jnp.zeros_like(counts_ref)
        acc_ref[0] = jnp.float32(0.0)

    f = f_ref[...]
    e = e_ref[...]
    fb = fb_ref[...]
    eb = eb_ref[...]
    s = jnp.dot(fb, eb, preferred_element_type=jnp.float32)
    rowsq = jnp.sum(f * f, axis=1, keepdims=True)
    colsq = jnp.sum(e * e, axis=0, keepdims=True)
    cw = 256
    nch = _NE // cw
    spill = frozenset(c for c in range(nch) if c % 4 == 3)
    mval = jnp.full((_MT, 1), jnp.inf, jnp.float32)
    mtrue = jnp.full((_MT, 1), jnp.inf, jnp.float32)
    midx = jnp.zeros((_MT, 1), jnp.int32)
    for c in range(nch):
        dd = (rowsq - 2.0 * s[:, c * cw:(c + 1) * cw]) + colsq[:, c * cw:(c + 1) * cw]
        mc = jnp.min(dd, axis=1, keepdims=True)
        ids_c = lax.broadcasted_iota(jnp.int32, dd.shape, 1) + jnp.int32(c * cw)
        ic = jnp.min(jnp.where(dd == mc, ids_c, jnp.int32(_NE)),
                     axis=1, keepdims=True)
        better = (mc < mval) | ((mc == mval) & (ic < midx))
        mval = jnp.where(better, mc, mval)
        midx = jnp.where(better, ic, midx)
        mtrue = jnp.minimum(mtrue, mc)
        if c in spill:
            mval = mval.astype(jnp.bfloat16).astype(jnp.float32)
    idx = midx[:, 0]
    idx_ref[0, 0, :] = idx
    acc_ref[0] += jnp.sum(mtrue)
    ids = lax.broadcasted_iota(jnp.int32, (_MT, _NE), 1)
    counts_ref[...] += jnp.sum((ids == idx[:, None]).astype(jnp.float32),
                               axis=0, keepdims=True)

    @pl.when(i == _NBLK - 1)
    def _fin():
        loss_ref[0, 0] = acc_ref[0] / jnp.float32(_TOKENS * _DIM)
        p = counts_ref[...] * jnp.float32(1.0 / _TOKENS)
        ent = jnp.sum(p * jnp.log(p + 1e-10))
        perp_ref[0, 0] = jnp.exp(-ent)


def _vq_tc(flatten, embed):
    fb = flatten.astype(jnp.bfloat16)
    eb = embed.astype(jnp.bfloat16)
    return pl.pallas_call(
        _vq_body,
        grid=(_NBLK,),
        in_specs=[
            pl.BlockSpec((_MT, _DIM), lambda i: (i, 0)),
            pl.BlockSpec((_DIM, _NE), lambda i: (0, 0)),
            pl.BlockSpec((_MT, _DIM), lambda i: (i, 0)),
            pl.BlockSpec((_DIM, _NE), lambda i: (0, 0)),
        ],
        out_specs=[
            pl.BlockSpec((1, 1, _MT), lambda i: (i, 0, 0)),
            pl.BlockSpec(memory_space=pltpu.SMEM),
            pl.BlockSpec(memory_space=pltpu.SMEM),
        ],
        out_shape=[
            jax.ShapeDtypeStruct((_NBLK, 1, _MT), jnp.int32),
            jax.ShapeDtypeStruct((1, 1), jnp.float32),
            jax.ShapeDtypeStruct((1, 1), jnp.float32),
        ],
        scratch_shapes=[
            pltpu.VMEM((1, _NE), jnp.float32),
            pltpu.SMEM((1,), jnp.float32),
        ],
    )(flatten, embed, fb, eb)


_NC = 2
_NS = 16
_NW = _NC * _NS
_ROWS_W = _TOKENS // _NW
_CHUNK = 256


def _gather_body(cb_ref, idx_ref, out_ref, idx_v, rows_v, sem):
    wid = lax.axis_index("s") * _NC + lax.axis_index("c")
    base = wid * _ROWS_W
    pltpu.sync_copy(idx_ref.at[pl.ds(base, _ROWS_W)], idx_v)
    for k in range(_ROWS_W // _CHUNK):
        pltpu.async_copy(
            cb_ref.at[idx_v.at[pl.ds(k * _CHUNK, _CHUNK)]], rows_v, sem
        ).wait()
        pltpu.sync_copy(rows_v, out_ref.at[pl.ds(base + k * _CHUNK, _CHUNK)])


def _sc_gather(codebook, idx):
    mesh = plsc.VectorSubcoreMesh(core_axis_name="c", subcore_axis_name="s")
    run = functools.partial(
        pl.kernel,
        mesh=mesh,
        out_type=jax.ShapeDtypeStruct((_TOKENS, _DIM), jnp.float32),
        scratch_types=[
            pltpu.VMEM((_ROWS_W,), jnp.int32),
            pltpu.VMEM((_CHUNK, _DIM), jnp.float32),
            pltpu.SemaphoreType.DMA,
        ],
    )(_gather_body)
    return run(codebook, idx)


def kernel(input, embed):
    flatten = input.reshape(-1, _DIM)
    idx3, loss, perp = _vq_tc(flatten, embed)
    codebook = embed.T
    q = _sc_gather(codebook, idx3.reshape(-1))
    return (q.reshape(input.shape), loss.reshape(()), perp.reshape(()))


# speedup vs baseline: 2.0632x; 2.0632x over previous
import functools

import jax
import jax.numpy as jnp
from jax import lax
from jax.experimental import pallas as pl
from jax.experimental.pallas import tpu as pltpu
from jax.experimental.pallas import tpu_sc as plsc

_DIM = 256
_NE = 8192
_TOKENS = 16384
_MT = 128
_NBLK = _TOKENS // _MT


def _vq_body(f_ref, fb_ref, eb_ref, csq_ref,
             idx_ref, loss_ref, perp_ref, counts_ref, acc_ref):
    i = pl.program_id(0)

    @pl.when(i == 0)
    def _init():
        counts_ref[...] = jnp.zeros_like(counts_ref)
        acc_ref[0] = jnp.float32(0.0)

    f = f_ref[...]
    fb = fb_ref[...]
    eb = eb_ref[...]
    s = jnp.dot(fb, eb, preferred_element_type=jnp.float32)
    rowsq = jnp.sum(f * f, axis=1, keepdims=True)
    colsq = csq_ref[...]
    cw = 1024
    nch = _NE // cw
    mval = jnp.full((_MT, 1), jnp.inf, jnp.float32)
    midx = jnp.zeros((_MT, 1), jnp.int32)
    for c in range(nch):
        dd = (rowsq - 2.0 * s[:, c * cw:(c + 1) * cw]) + colsq[:, c * cw:(c + 1) * cw]
        mc = jnp.min(dd, axis=1, keepdims=True)
        ids_c = lax.broadcasted_iota(jnp.int32, dd.shape, 1) + jnp.int32(c * cw)
        ic = jnp.min(jnp.where(dd == mc, ids_c, jnp.int32(_NE)),
                     axis=1, keepdims=True)
        better = (mc < mval) | ((mc == mval) & (ic < midx))
        mval = jnp.where(better, mc, mval).astype(jnp.bfloat16).astype(jnp.float32)
        midx = jnp.where(better, ic, midx)
    idx = midx[:, 0]
    idx_ref[0, 0, :] = idx
    acc_ref[0] += jnp.sum(mval)
    ids = lax.broadcasted_iota(jnp.int32, (_MT, _NE), 1)
    counts_ref[...] += jnp.sum((ids == idx[:, None]).astype(jnp.float32),
                               axis=0, keepdims=True)

    @pl.when(i == _NBLK - 1)
    def _fin():
        loss_ref[0, 0] = acc_ref[0] / jnp.float32(_TOKENS * _DIM)
        p = counts_ref[...] * jnp.float32(1.0 / _TOKENS)
        ent = jnp.sum(p * jnp.log(p + 1e-10))
        perp_ref[0, 0] = jnp.exp(-ent)


def _vq_tc(flatten, embed):
    fb = flatten.astype(jnp.bfloat16)
    eb = embed.astype(jnp.bfloat16)
    colsq = (embed ** 2).sum(0, keepdims=True)
    return pl.pallas_call(
        _vq_body,
        grid=(_NBLK,),
        in_specs=[
            pl.BlockSpec((_MT, _DIM), lambda i: (i, 0)),
            pl.BlockSpec((_MT, _DIM), lambda i: (i, 0)),
            pl.BlockSpec((_DIM, _NE), lambda i: (0, 0)),
            pl.BlockSpec((1, _NE), lambda i: (0, 0)),
        ],
        out_specs=[
            pl.BlockSpec((1, 1, _MT), lambda i: (i, 0, 0)),
            pl.BlockSpec(memory_space=pltpu.SMEM),
            pl.BlockSpec(memory_space=pltpu.SMEM),
        ],
        out_shape=[
            jax.ShapeDtypeStruct((_NBLK, 1, _MT), jnp.int32),
            jax.ShapeDtypeStruct((1, 1), jnp.float32),
            jax.ShapeDtypeStruct((1, 1), jnp.float32),
        ],
        scratch_shapes=[
            pltpu.VMEM((1, _NE), jnp.float32),
            pltpu.SMEM((1,), jnp.float32),
        ],
    )(flatten, fb, eb, colsq)


_NC = 2
_NS = 16
_NW = _NC * _NS
_ROWS_W = _TOKENS // _NW
_CHUNK = 256


def _gather_body(cb_ref, idx_ref, out_ref, idx_v, rows_v, sem):
    wid = lax.axis_index("s") * _NC + lax.axis_index("c")
    base = wid * _ROWS_W
    pltpu.sync_copy(idx_ref.at[pl.ds(base, _ROWS_W)], idx_v)
    for k in range(_ROWS_W // _CHUNK):
        pltpu.async_copy(
            cb_ref.at[idx_v.at[pl.ds(k * _CHUNK, _CHUNK)]], rows_v, sem
        ).wait()
        pltpu.sync_copy(rows_v, out_ref.at[pl.ds(base + k * _CHUNK, _CHUNK)])


def _sc_gather(codebook, idx):
    mesh = plsc.VectorSubcoreMesh(core_axis_name="c", subcore_axis_name="s")
    run = functools.partial(
        pl.kernel,
        mesh=mesh,
        out_type=jax.ShapeDtypeStruct((_TOKENS, _DIM), jnp.float32),
        scratch_types=[
            pltpu.VMEM((_ROWS_W,), jnp.int32),
            pltpu.VMEM((_CHUNK, _DIM), jnp.float32),
            pltpu.SemaphoreType.DMA,
        ],
    )(_gather_body)
    return run(codebook, idx)


def kernel(input, embed):
    flatten = input.reshape(-1, _DIM)
    idx3, loss, perp = _vq_tc(flatten, embed)
    codebook = embed.T
    q = _sc_gather(codebook, idx3.reshape(-1))
    return (q.reshape(input.shape), loss.reshape(()), perp.reshape(()))


# speedup vs baseline: 2.1982x; 1.0654x over previous
import functools

import jax
import jax.numpy as jnp
from jax import lax
from jax.experimental import pallas as pl
from jax.experimental.pallas import tpu as pltpu
from jax.experimental.pallas import tpu_sc as plsc

_DIM = 256
_NE = 8192
_TOKENS = 16384
_MT = 256
_NBLK = _TOKENS // _MT


def _vq_body(f_ref, fb_ref, eb_ref, csq_ref,
             idx_ref, loss_ref, perp_ref, counts_ref, acc_ref):
    i = pl.program_id(0)

    @pl.when(i == 0)
    def _init():
        counts_ref[...] = jnp.zeros_like(counts_ref)
        acc_ref[0] = jnp.float32(0.0)

    f = f_ref[...]
    fb = fb_ref[...]
    eb = eb_ref[...]
    s = jnp.dot(fb, eb, preferred_element_type=jnp.float32)
    rowsq = jnp.sum(f * f, axis=1, keepdims=True)
    colsq = csq_ref[...]
    cw = 1024
    nch = _NE // cw
    mval = jnp.full((_MT, 1), jnp.inf, jnp.float32)
    midx = jnp.zeros((_MT, 1), jnp.int32)
    for c in range(nch):
        dd = (rowsq - 2.0 * s[:, c * cw:(c + 1) * cw]) + colsq[:, c * cw:(c + 1) * cw]
        mc = jnp.min(dd, axis=1, keepdims=True)
        ids_c = lax.broadcasted_iota(jnp.int32, dd.shape, 1) + jnp.int32(c * cw)
        ic = jnp.min(jnp.where(dd == mc, ids_c, jnp.int32(_NE)),
                     axis=1, keepdims=True)
        better = (mc < mval) | ((mc == mval) & (ic < midx))
        mval = jnp.where(better, mc, mval).astype(jnp.bfloat16).astype(jnp.float32)
        midx = jnp.where(better, ic, midx)
    idx = midx[:, 0]
    idx_ref[0, 0, :] = idx
    acc_ref[0] += jnp.sum(mval)
    ids = lax.broadcasted_iota(jnp.int32, (_MT, _NE), 1)
    counts_ref[...] += jnp.sum((ids == idx[:, None]).astype(jnp.float32),
                               axis=0, keepdims=True)

    @pl.when(i == _NBLK - 1)
    def _fin():
        loss_ref[0, 0] = acc_ref[0] / jnp.float32(_TOKENS * _DIM)
        p = counts_ref[...] * jnp.float32(1.0 / _TOKENS)
        ent = jnp.sum(p * jnp.log(p + 1e-10))
        perp_ref[0, 0] = jnp.exp(-ent)


def _vq_tc(flatten, embed):
    fb = flatten.astype(jnp.bfloat16)
    eb = embed.astype(jnp.bfloat16)
    colsq = (embed ** 2).sum(0, keepdims=True)
    return pl.pallas_call(
        _vq_body,
        grid=(_NBLK,),
        in_specs=[
            pl.BlockSpec((_MT, _DIM), lambda i: (i, 0)),
            pl.BlockSpec((_MT, _DIM), lambda i: (i, 0)),
            pl.BlockSpec((_DIM, _NE), lambda i: (0, 0)),
            pl.BlockSpec((1, _NE), lambda i: (0, 0)),
        ],
        out_specs=[
            pl.BlockSpec((1, 1, _MT), lambda i: (i, 0, 0)),
            pl.BlockSpec(memory_space=pltpu.SMEM),
            pl.BlockSpec(memory_space=pltpu.SMEM),
        ],
        out_shape=[
            jax.ShapeDtypeStruct((_NBLK, 1, _MT), jnp.int32),
            jax.ShapeDtypeStruct((1, 1), jnp.float32),
            jax.ShapeDtypeStruct((1, 1), jnp.float32),
        ],
        scratch_shapes=[
            pltpu.VMEM((1, _NE), jnp.float32),
            pltpu.SMEM((1,), jnp.float32),
        ],
    )(flatten, fb, eb, colsq)


_NC = 2
_NS = 16
_NW = _NC * _NS
_ROWS_W = _TOKENS // _NW
_CHUNK = 256


def _gather_body(cb_ref, idx_ref, out_ref, idx_v, rows_v, sem):
    wid = lax.axis_index("s") * _NC + lax.axis_index("c")
    base = wid * _ROWS_W
    pltpu.sync_copy(idx_ref.at[pl.ds(base, _ROWS_W)], idx_v)
    for k in range(_ROWS_W // _CHUNK):
        pltpu.async_copy(
            cb_ref.at[idx_v.at[pl.ds(k * _CHUNK, _CHUNK)]], rows_v, sem
        ).wait()
        pltpu.sync_copy(rows_v, out_ref.at[pl.ds(base + k * _CHUNK, _CHUNK)])


def _sc_gather(codebook, idx):
    mesh = plsc.VectorSubcoreMesh(core_axis_name="c", subcore_axis_name="s")
    run = functools.partial(
        pl.kernel,
        mesh=mesh,
        out_type=jax.ShapeDtypeStruct((_TOKENS, _DIM), jnp.float32),
        scratch_types=[
            pltpu.VMEM((_ROWS_W,), jnp.int32),
            pltpu.VMEM((_CHUNK, _DIM), jnp.float32),
            pltpu.SemaphoreType.DMA,
        ],
    )(_gather_body)
    return run(codebook, idx)


def kernel(input, embed):
    flatten = input.reshape(-1, _DIM)
    idx3, loss, perp = _vq_tc(flatten, embed)
    codebook = embed.T
    q = _sc_gather(codebook, idx3.reshape(-1))
    return (q.reshape(input.shape), loss.reshape(()), perp.reshape(()))
